# baseline (speedup 1.0000x reference)
import functools
from functools import partial

import jax
import jax.numpy as jnp
from jax import lax
from jax.experimental import pallas as pl
from jax.experimental.pallas import tpu as pltpu
from jax.experimental.pallas import tpu_sc as plsc

_NC, _NS, _L = 2, 16, 16
_NW = _NC * _NS


def _capacity(S, E, top_k=2, cap_f=1.0, min_cap=8):
    return max(min_cap, int(S * top_k * cap_f / E))


def _router_body(x_ref, gw_ref, gb_ref, idx_ref, gate_ref, cnt_ref,
                 *, S, E, C, G):
    g = pl.program_id(0)
    x = x_ref[0]
    gw = gw_ref[...]
    logits = jnp.dot(x, gw, preferred_element_type=jnp.float32)
    logits = logits + gb_ref[...]
    m = jnp.max(logits, axis=1, keepdims=True)
    ex = jnp.exp(logits - m)
    probs = ex / jnp.sum(ex, axis=1, keepdims=True)

    lane = lax.broadcasted_iota(jnp.int32, (S, E), 1)
    m0 = jnp.max(probs, axis=1, keepdims=True)
    e0 = jnp.min(jnp.where(probs == m0, lane, E), axis=1, keepdims=True)
    probs1 = jnp.where(lane == e0, -1.0, probs)
    m1 = jnp.max(probs1, axis=1, keepdims=True)
    e1 = jnp.min(jnp.where(probs1 == m1, lane, E), axis=1, keepdims=True)

    row = lax.broadcasted_iota(jnp.int32, (S, S), 0)
    col = lax.broadcasted_iota(jnp.int32, (S, S), 1)
    tril = (row >= col).astype(jnp.float32)
    oh0 = (lane == e0).astype(jnp.float32)
    oh1 = (lane == e1).astype(jnp.float32)
    pos0m = jnp.dot(tril, oh0, preferred_element_type=jnp.float32)
    pos1m = jnp.dot(tril, oh1, preferred_element_type=jnp.float32)
    pos0 = jnp.sum(jnp.where(lane == e0, pos0m, 0.0), axis=1, keepdims=True)
    pos1 = jnp.sum(jnp.where(lane == e1, pos1m, 0.0), axis=1, keepdims=True)
    pos0 = pos0.astype(jnp.int32)
    pos1 = pos1.astype(jnp.int32)

    NSLOT = G * E * C
    valid0 = pos0 < C
    valid1 = pos1 < C
    ds0 = jnp.where(valid0, (e0 * G + g) * C + pos0 - 1, NSLOT)
    ds1 = jnp.where(valid1, (e1 * G + g) * C + pos1 - 1, NSLOT)
    g0 = jnp.where(valid0, m0, 0.0)
    g1 = jnp.where(valid1, m1, 0.0)
    cg0 = jnp.where(valid0, ds0, 0)
    cg1 = jnp.where(valid1, ds1, 0)

    zi = jnp.zeros((S, E), jnp.int32)
    idx_ref[0] = (jnp.where(lane == 0, ds0, zi) + jnp.where(lane == 1, ds1, zi)
                  + jnp.where(lane == 2, cg0, zi) + jnp.where(lane == 3, cg1, zi))
    zf = jnp.zeros((S, E), jnp.float32)
    gate_ref[0] = jnp.where(lane == 0, g0, zf) + jnp.where(lane == 1, g1, zf)

    n0 = jnp.minimum(jnp.sum(oh0, axis=0, keepdims=True),
                     float(C - 1)).astype(jnp.int32)
    n1 = jnp.minimum(jnp.sum(oh1, axis=0, keepdims=True),
                     float(C - 1)).astype(jnp.int32)
    n0t = jnp.concatenate([n0, n0, n0, n0], axis=1)
    n1t = jnp.concatenate([n1, n1, n1, n1], axis=1)
    W4 = 4 * E
    riota = lax.broadcasted_iota(jnp.int32, (8, W4), 0)
    ciota = lax.broadcasted_iota(jnp.int32, (8, W4), 1)
    mine0 = (ciota >= g * E) & (ciota < (g + 1) * E)
    mine1 = (ciota >= (G + g) * E) & (ciota < (G + g + 1) * E)
    val = (jnp.where(mine0, jnp.broadcast_to(n0t, (8, W4)), 0)
           + jnp.where(mine1, jnp.broadcast_to(n1t, (8, W4)), 0))
    keep = (riota == 0) & (mine0 | mine1)
    cnt_ref[...] = jnp.where(keep, val, cnt_ref[...])


def _run_router(xg, gate_weight, gate_bias, *, interpret=False):
    G, S, D = xg.shape
    E = gate_weight.shape[1]
    C = _capacity(S, E)
    body = partial(_router_body, S=S, E=E, C=C, G=G)
    idx, gates, cnt = pl.pallas_call(
        body,
        grid=(G,),
        in_specs=[
            pl.BlockSpec((1, S, D), lambda g: (g, 0, 0)),
            pl.BlockSpec((D, E), lambda g: (0, 0)),
            pl.BlockSpec((1, E), lambda g: (0, 0)),
        ],
        out_specs=[
            pl.BlockSpec((1, S, E), lambda g: (g, 0, 0)),
            pl.BlockSpec((1, S, E), lambda g: (g, 0, 0)),
            pl.BlockSpec((8, 4 * E), lambda g: (0, 0)),
        ],
        out_shape=[
            jax.ShapeDtypeStruct((G, S, E), jnp.int32),
            jax.ShapeDtypeStruct((G, S, E), jnp.float32),
            jax.ShapeDtypeStruct((8, 4 * E), jnp.int32),
        ],
        interpret=interpret,
    )(xg, gate_weight, gate_bias.reshape(1, E))
    return idx, gates, cnt


def _scatter_dispatch_sc(x_flat, dsidx, *, NSLOT, interpret=False):
    N, D = x_flat.shape
    per_w = N // _NW
    chunk = 32
    nch = per_w // chunk
    OUT = NSLOT + 1024
    mesh = plsc.VectorSubcoreMesh(
        core_axis_name="c", subcore_axis_name="s",
        num_cores=_NC, num_subcores=_NS)

    @functools.partial(
        pl.kernel, mesh=mesh, interpret=interpret,
        compiler_params=pltpu.CompilerParams(needs_layout_passes=False),
        out_type=(jax.ShapeDtypeStruct((OUT, D), jnp.float32),
                  jax.ShapeDtypeStruct((OUT, D), jnp.float32)),
        scratch_types=[
            pltpu.VMEM((2, nch, chunk), jnp.int32),
            pltpu.VMEM((chunk, D), jnp.float32),
            pltpu.VMEM((chunk, D), jnp.float32),
            pltpu.SemaphoreType.DMA,
            pltpu.SemaphoreType.DMA,
            pltpu.SemaphoreType.DMA,
            pltpu.SemaphoreType.DMA,
        ],
    )
    def k(x_hbm, ds_hbm, out0_hbm, out1_hbm, idx_v, ra_v, rb_v,
          sra, srb, ssa, ssb):
        wid = lax.axis_index("s") * _NC + lax.axis_index("c")
        t0 = wid * per_w
        pltpu.sync_copy(ds_hbm.at[wid], idx_v)

        sets = ((ra_v, sra, ssa), (rb_v, srb, ssb))

        def start_read(j, buf, sr):
            pltpu.async_copy(x_hbm.at[pl.ds(t0 + j * chunk, chunk)], buf, sr)

        def drain_read(buf, sr):
            pltpu.make_async_copy(x_hbm.at[pl.ds(0, chunk)], buf, sr).wait()

        def drain_scatter(buf, ss):
            pltpu.make_async_copy(buf, out0_hbm.at[pl.ds(0, chunk)],
                                  ss).wait()
            pltpu.make_async_copy(buf, out1_hbm.at[pl.ds(0, chunk)],
                                  ss).wait()

        def do_chunk(j, buf, sr, ss):
            drain_read(buf, sr)
            pltpu.async_copy(buf, out0_hbm.at[idx_v.at[0, j]], ss)
            pltpu.async_copy(buf, out1_hbm.at[idx_v.at[1, j]], ss)

        start_read(0, ra_v, sra)

        def pipe(jj, _):
            for p, (buf, sr, ss) in enumerate(sets):
                j = jj + p
                nxt = sets[1 - p]

                @pl.when(j + 1 < nch)
                def _prefetch():
                    @pl.when(j >= 1)
                    def _reclaim():
                        drain_scatter(nxt[0], nxt[2])
                    start_read(j + 1, nxt[0], nxt[1])

                do_chunk(j, buf, sr, ss)
            return ()
        lax.fori_loop(0, nch // 2, lambda q, c: pipe(q * 2, c), ())
        drain_scatter(ra_v, ssa)
        drain_scatter(rb_v, ssb)

    return k(x_flat, dsidx)


def _expert_ffn_body(cnt_sref, x0_ref, x1_ref, wk_ref, wv_ref, o_ref,
                     *, C, G, E):
    i = pl.program_id(0)
    e = i // G
    g = lax.rem(i, G)
    c = lax.broadcasted_iota(jnp.int32, (C, 1), 0)
    thr0 = cnt_sref[g * E + e]
    thr1 = cnt_sref[G * E + g * E + e]
    x = (jnp.where(c < thr0, x0_ref[...], 0.0)
         + jnp.where(c < thr1, x1_ref[...], 0.0))
    h = jnp.dot(x, wk_ref[0], preferred_element_type=jnp.float32)
    h = jax.nn.gelu(h)
    o_ref[...] = jnp.dot(h, wv_ref[0], preferred_element_type=jnp.float32)


def _run_expert_ffn(cnt32, xin0, xin1, wk, wv, *, G, C, interpret=False):
    D = xin0.shape[1]
    E, _, H = wk.shape
    NSLOT = E * G * C
    body = partial(_expert_ffn_body, C=C, G=G, E=E)
    out = pl.pallas_call(
        body,
        grid_spec=pltpu.PrefetchScalarGridSpec(
            num_scalar_prefetch=1,
            grid=(E * G,),
            in_specs=[
                pl.BlockSpec((C, D), lambda i, c: (i, 0)),
                pl.BlockSpec((C, D), lambda i, c: (i, 0)),
                pl.BlockSpec((1, D, H), lambda i, c: (i // G, 0, 0)),
                pl.BlockSpec((1, H, D), lambda i, c: (i // G, 0, 0)),
            ],
            out_specs=pl.BlockSpec((C, D), lambda i, c: (i, 0)),
        ),
        out_shape=jax.ShapeDtypeStruct((NSLOT, D), jnp.float32),
        interpret=interpret,
    )(cnt32, xin0, xin1, wk, wv)
    return out


def _ffn_body(x_ref, wk_ref, wv_ref, o_ref):
    h = jnp.dot(x_ref[...], wk_ref[0], preferred_element_type=jnp.float32)
    h = jax.nn.gelu(h)
    o_ref[...] = jnp.dot(h, wv_ref[0], preferred_element_type=jnp.float32)


def _run_ffn(xin, wk, wv, rows_per_block, *, interpret=False):
    N, D = xin.shape
    W, _, H = wk.shape
    nblocks = N // rows_per_block
    per_w = nblocks // W
    out = pl.pallas_call(
        _ffn_body,
        grid=(nblocks,),
        in_specs=[
            pl.BlockSpec((rows_per_block, D), lambda i: (i, 0)),
            pl.BlockSpec((1, D, H), lambda i: (i // per_w, 0, 0)),
            pl.BlockSpec((1, H, D), lambda i: (i // per_w, 0, 0)),
        ],
        out_specs=pl.BlockSpec((rows_per_block, D), lambda i: (i, 0)),
        out_shape=jax.ShapeDtypeStruct((N, D), jnp.float32),
        interpret=interpret,
    )(xin, wk, wv)
    return out


def _combine_sc(yexp, ysh, cg, gates, *, interpret=False):
    N, D = ysh.shape
    per_w = N // _NW
    chunk = 8
    R = 4
    mesh = plsc.VectorSubcoreMesh(
        core_axis_name="c", subcore_axis_name="s",
        num_cores=_NC, num_subcores=_NS)

    ring_scratch = []
    for _ in range(R):
        ring_scratch += [pltpu.VMEM((chunk, D), jnp.float32)] * 3
        ring_scratch += [pltpu.SemaphoreType.DMA] * 2

    @functools.partial(
        pl.kernel, mesh=mesh, interpret=interpret,
        compiler_params=pltpu.CompilerParams(needs_layout_passes=False),
        out_type=jax.ShapeDtypeStruct((N, D), jnp.float32),
        scratch_types=[
            pltpu.VMEM((per_w,), jnp.int32),
            pltpu.VMEM((per_w,), jnp.int32),
            pltpu.VMEM((per_w,), jnp.float32),
            pltpu.VMEM((per_w,), jnp.float32),
        ] + ring_scratch,
    )
    def k(yexp_hbm, ysh_hbm, cg_hbm, gates_hbm, out_hbm,
          i0_v, i1_v, g0_v, g1_v, *ring):
        wid = lax.axis_index("s") * _NC + lax.axis_index("c")
        t0 = wid * per_w
        nch = per_w // chunk
        pltpu.sync_copy(cg_hbm.at[0, pl.ds(t0, per_w)], i0_v)
        pltpu.sync_copy(cg_hbm.at[1, pl.ds(t0, per_w)], i1_v)
        pltpu.sync_copy(gates_hbm.at[0, pl.ds(t0, per_w)], g0_v)
        pltpu.sync_copy(gates_hbm.at[1, pl.ds(t0, per_w)], g1_v)

        sets = tuple(tuple(ring[5 * q:5 * q + 5]) for q in range(R))

        def start_fetch(j, y0, y1, acc, sg):
            b = j * chunk
            pltpu.async_copy(yexp_hbm.at[i0_v.at[pl.ds(b, chunk)]], y0, sg)
            pltpu.async_copy(yexp_hbm.at[i1_v.at[pl.ds(b, chunk)]], y1, sg)
            pltpu.async_copy(ysh_hbm.at[pl.ds(t0 + b, chunk)], acc, sg)

        def drain_fetch(y0, y1, acc, sg):
            pltpu.make_async_copy(ysh_hbm.at[pl.ds(0, chunk)], y0, sg).wait()
            pltpu.make_async_copy(ysh_hbm.at[pl.ds(0, chunk)], y1, sg).wait()
            pltpu.make_async_copy(ysh_hbm.at[pl.ds(0, chunk)], acc, sg).wait()

        def drain_out(acc, so):
            pltpu.make_async_copy(
                acc, out_hbm.at[pl.ds(t0, chunk)], so).wait()

        def do_chunk(j, y0, y1, acc, sg, so):
            drain_fetch(y0, y1, acc, sg)
            b = j * chunk

            def tok(i, _):
                s0 = plsc.load_gather(g0_v, [jnp.full((_L,), b + i, jnp.int32)])
                s1 = plsc.load_gather(g1_v, [jnp.full((_L,), b + i, jnp.int32)])

                def vec(w, _):
                    sl = pl.ds(w * _L, _L)
                    acc[i, sl] = (acc[i, sl] + s0 * y0[i, sl]
                                  + s1 * y1[i, sl])
                    return ()
                lax.fori_loop(0, D // _L, vec, (), unroll=8)
                return ()
            lax.fori_loop(0, chunk, tok, ())
            pltpu.async_copy(acc, out_hbm.at[pl.ds(t0 + b, chunk)], so)

        for q in range(R - 1):
            start_fetch(q, sets[q][0], sets[q][1], sets[q][2], sets[q][3])

        def pipe(jj, _):
            for p, (y0, y1, acc, sg, so) in enumerate(sets):
                j = jj + p
                nxt = sets[(p + R - 1) % R]

                @pl.when(j + R - 1 < nch)
                def _prefetch():
                    @pl.when(j >= 1)
                    def _reclaim():
                        drain_out(nxt[2], nxt[4])
                    start_fetch(j + R - 1, nxt[0], nxt[1], nxt[2], nxt[3])

                do_chunk(j, y0, y1, acc, sg, so)
            return ()
        lax.fori_loop(0, nch // R, lambda q, c: pipe(q * R, c), ())
        for q in range(R):
            drain_out(sets[q][2], sets[q][4])

    return k(yexp, ysh, cg, gates)


def kernel(x, gate_weight, gate_bias, ff_keys, ff_values,
           shared_keys, shared_values):
    B, S_in, D = x.shape
    E = gate_weight.shape[1]
    group_size = min(S_in, 4096)
    G = (B * S_in) // group_size
    S = group_size
    C = _capacity(S, E)
    N = G * S
    NSLOT = G * E * C

    xg = x.reshape(G, S, D)
    idx, gates, cnt = _run_router(xg, gate_weight, gate_bias)

    per_w = N // _NW
    chunk = 32
    nch = per_w // chunk
    ds = jnp.transpose(idx[:, :, 0:2].reshape(_NW, nch, chunk, 2),
                       (0, 3, 1, 2))
    cg = idx[:, :, 2:4].reshape(N, 2).T
    gk = gates[:, :, 0:2].reshape(N, 2).T
    cnt32 = cnt[0]

    x_flat = x.reshape(N, D)
    xin0, xin1 = _scatter_dispatch_sc(x_flat, ds, NSLOT=NSLOT)

    yexp = _run_expert_ffn(cnt32, xin0, xin1, ff_keys, ff_values, G=G, C=C)
    ysh = _run_ffn(x_flat, shared_keys, shared_values, rows_per_block=512)

    out = _combine_sc(yexp, ysh, cg, gk)
    return out.reshape(B, S_in, D)


# speedup vs baseline: 1.0020x; 1.0020x over previous
import functools
from functools import partial

import jax
import jax.numpy as jnp
from jax import lax
from jax.experimental import pallas as pl
from jax.experimental.pallas import tpu as pltpu
from jax.experimental.pallas import tpu_sc as plsc

_NC, _NS, _L = 2, 16, 16
_NW = _NC * _NS


def _capacity(S, E, top_k=2, cap_f=1.0, min_cap=8):
    return max(min_cap, int(S * top_k * cap_f / E))


def _router_body(x_ref, gw_ref, gb_ref, idx_ref, gate_ref, cnt_ref,
                 *, S, E, C, G):
    g = pl.program_id(0)
    x = x_ref[0]
    gw = gw_ref[...]
    logits = jnp.dot(x, gw, preferred_element_type=jnp.float32)
    logits = logits + gb_ref[...]
    m = jnp.max(logits, axis=1, keepdims=True)
    ex = jnp.exp(logits - m)
    probs = ex / jnp.sum(ex, axis=1, keepdims=True)

    lane = lax.broadcasted_iota(jnp.int32, (S, E), 1)
    m0 = jnp.max(probs, axis=1, keepdims=True)
    e0 = jnp.min(jnp.where(probs == m0, lane, E), axis=1, keepdims=True)
    probs1 = jnp.where(lane == e0, -1.0, probs)
    m1 = jnp.max(probs1, axis=1, keepdims=True)
    e1 = jnp.min(jnp.where(probs1 == m1, lane, E), axis=1, keepdims=True)

    row = lax.broadcasted_iota(jnp.int32, (S, S), 0)
    col = lax.broadcasted_iota(jnp.int32, (S, S), 1)
    tril = (row >= col).astype(jnp.float32)
    oh0 = (lane == e0).astype(jnp.float32)
    oh1 = (lane == e1).astype(jnp.float32)
    pos0m = jnp.dot(tril, oh0, preferred_element_type=jnp.float32)
    pos1m = jnp.dot(tril, oh1, preferred_element_type=jnp.float32)
    pos0 = jnp.sum(jnp.where(lane == e0, pos0m, 0.0), axis=1, keepdims=True)
    pos1 = jnp.sum(jnp.where(lane == e1, pos1m, 0.0), axis=1, keepdims=True)
    pos0 = pos0.astype(jnp.int32)
    pos1 = pos1.astype(jnp.int32)

    NSLOT = G * E * C
    valid0 = pos0 < C
    valid1 = pos1 < C
    ds0 = jnp.where(valid0, (e0 * G + g) * C + pos0 - 1, NSLOT)
    ds1 = jnp.where(valid1, (e1 * G + g) * C + pos1 - 1, NSLOT)
    g0 = jnp.where(valid0, m0, 0.0)
    g1 = jnp.where(valid1, m1, 0.0)
    cg0 = jnp.where(valid0, ds0, 0)
    cg1 = jnp.where(valid1, ds1, 0)

    zi = jnp.zeros((S, E), jnp.int32)
    idx_ref[0] = (jnp.where(lane == 0, ds0, zi) + jnp.where(lane == 1, ds1, zi)
                  + jnp.where(lane == 2, cg0, zi) + jnp.where(lane == 3, cg1, zi))
    zf = jnp.zeros((S, E), jnp.float32)
    gate_ref[0] = jnp.where(lane == 0, g0, zf) + jnp.where(lane == 1, g1, zf)

    n0 = jnp.minimum(jnp.sum(oh0, axis=0, keepdims=True),
                     float(C - 1)).astype(jnp.int32)
    n1 = jnp.minimum(jnp.sum(oh1, axis=0, keepdims=True),
                     float(C - 1)).astype(jnp.int32)
    n0t = jnp.concatenate([n0, n0, n0, n0], axis=1)
    n1t = jnp.concatenate([n1, n1, n1, n1], axis=1)
    W4 = 4 * E
    riota = lax.broadcasted_iota(jnp.int32, (8, W4), 0)
    ciota = lax.broadcasted_iota(jnp.int32, (8, W4), 1)
    mine0 = (ciota >= g * E) & (ciota < (g + 1) * E)
    mine1 = (ciota >= (G + g) * E) & (ciota < (G + g + 1) * E)
    val = (jnp.where(mine0, jnp.broadcast_to(n0t, (8, W4)), 0)
           + jnp.where(mine1, jnp.broadcast_to(n1t, (8, W4)), 0))
    keep = (riota == 0) & (mine0 | mine1)
    cnt_ref[...] = jnp.where(keep, val, cnt_ref[...])


def _run_router(xg, gate_weight, gate_bias, *, interpret=False):
    G, S, D = xg.shape
    E = gate_weight.shape[1]
    C = _capacity(S, E)
    body = partial(_router_body, S=S, E=E, C=C, G=G)
    idx, gates, cnt = pl.pallas_call(
        body,
        grid=(G,),
        in_specs=[
            pl.BlockSpec((1, S, D), lambda g: (g, 0, 0)),
            pl.BlockSpec((D, E), lambda g: (0, 0)),
            pl.BlockSpec((1, E), lambda g: (0, 0)),
        ],
        out_specs=[
            pl.BlockSpec((1, S, E), lambda g: (g, 0, 0)),
            pl.BlockSpec((1, S, E), lambda g: (g, 0, 0)),
            pl.BlockSpec((8, 4 * E), lambda g: (0, 0)),
        ],
        out_shape=[
            jax.ShapeDtypeStruct((G, S, E), jnp.int32),
            jax.ShapeDtypeStruct((G, S, E), jnp.float32),
            jax.ShapeDtypeStruct((8, 4 * E), jnp.int32),
        ],
        interpret=interpret,
    )(xg, gate_weight, gate_bias.reshape(1, E))
    return idx, gates, cnt


def _scatter_dispatch_sc(x_flat, dsidx, *, NSLOT, interpret=False):
    N, D = x_flat.shape
    per_w = N // _NW
    chunk = 32
    nch = per_w // chunk
    OUT = NSLOT + 1024
    mesh = plsc.VectorSubcoreMesh(
        core_axis_name="c", subcore_axis_name="s",
        num_cores=_NC, num_subcores=_NS)

    @functools.partial(
        pl.kernel, mesh=mesh, interpret=interpret,
        compiler_params=pltpu.CompilerParams(needs_layout_passes=False),
        out_type=(jax.ShapeDtypeStruct((OUT, D), jnp.float32),
                  jax.ShapeDtypeStruct((OUT, D), jnp.float32)),
        scratch_types=[
            pltpu.VMEM((2, nch, chunk), jnp.int32),
            pltpu.VMEM((chunk, D), jnp.float32),
            pltpu.VMEM((chunk, D), jnp.float32),
            pltpu.SemaphoreType.DMA,
            pltpu.SemaphoreType.DMA,
            pltpu.SemaphoreType.DMA,
            pltpu.SemaphoreType.DMA,
        ],
    )
    def k(x_hbm, ds_hbm, out0_hbm, out1_hbm, idx_v, ra_v, rb_v,
          sra, srb, ssa, ssb):
        wid = lax.axis_index("s") * _NC + lax.axis_index("c")
        t0 = wid * per_w
        pltpu.sync_copy(ds_hbm.at[wid], idx_v)

        sets = ((ra_v, sra, ssa), (rb_v, srb, ssb))

        def start_read(j, buf, sr):
            pltpu.async_copy(x_hbm.at[pl.ds(t0 + j * chunk, chunk)], buf, sr)

        def drain_read(buf, sr):
            pltpu.make_async_copy(x_hbm.at[pl.ds(0, chunk)], buf, sr).wait()

        def drain_scatter(buf, ss):
            pltpu.make_async_copy(buf, out0_hbm.at[pl.ds(0, chunk)],
                                  ss).wait()
            pltpu.make_async_copy(buf, out1_hbm.at[pl.ds(0, chunk)],
                                  ss).wait()

        def do_chunk(j, buf, sr, ss):
            drain_read(buf, sr)
            pltpu.async_copy(buf, out0_hbm.at[idx_v.at[0, j]], ss)
            pltpu.async_copy(buf, out1_hbm.at[idx_v.at[1, j]], ss)

        start_read(0, ra_v, sra)

        def pipe(jj, _):
            for p, (buf, sr, ss) in enumerate(sets):
                j = jj + p
                nxt = sets[1 - p]

                @pl.when(j + 1 < nch)
                def _prefetch():
                    @pl.when(j >= 1)
                    def _reclaim():
                        drain_scatter(nxt[0], nxt[2])
                    start_read(j + 1, nxt[0], nxt[1])

                do_chunk(j, buf, sr, ss)
            return ()
        lax.fori_loop(0, nch // 2, lambda q, c: pipe(q * 2, c), ())
        drain_scatter(ra_v, ssa)
        drain_scatter(rb_v, ssb)

    return k(x_flat, dsidx)


def _expert_ffn_body(cnt_sref, x0_ref, x1_ref, wk_ref, wv_ref, o_ref,
                     *, C, G, E):
    i = pl.program_id(0)
    e = i // G
    g = lax.rem(i, G)
    c = lax.broadcasted_iota(jnp.int32, (C, 1), 0)
    thr0 = cnt_sref[g * E + e]
    thr1 = cnt_sref[G * E + g * E + e]
    x = (jnp.where(c < thr0, x0_ref[...], 0.0)
         + jnp.where(c < thr1, x1_ref[...], 0.0))
    h = jnp.dot(x, wk_ref[0], preferred_element_type=jnp.float32)
    h = jax.nn.gelu(h)
    o_ref[...] = jnp.dot(h, wv_ref[0], preferred_element_type=jnp.float32)


def _run_expert_ffn(cnt32, xin0, xin1, wk, wv, *, G, C, interpret=False):
    D = xin0.shape[1]
    E, _, H = wk.shape
    NSLOT = E * G * C
    body = partial(_expert_ffn_body, C=C, G=G, E=E)
    out = pl.pallas_call(
        body,
        grid_spec=pltpu.PrefetchScalarGridSpec(
            num_scalar_prefetch=1,
            grid=(E * G,),
            in_specs=[
                pl.BlockSpec((C, D), lambda i, c: (i, 0)),
                pl.BlockSpec((C, D), lambda i, c: (i, 0)),
                pl.BlockSpec((1, D, H), lambda i, c: (i // G, 0, 0)),
                pl.BlockSpec((1, H, D), lambda i, c: (i // G, 0, 0)),
            ],
            out_specs=pl.BlockSpec((C, D), lambda i, c: (i, 0)),
        ),
        out_shape=jax.ShapeDtypeStruct((NSLOT, D), jnp.float32),
        interpret=interpret,
    )(cnt32, xin0, xin1, wk, wv)
    return out


def _ffn_body(x_ref, wk_ref, wv_ref, o_ref):
    h = jnp.dot(x_ref[...], wk_ref[0], preferred_element_type=jnp.float32)
    h = jax.nn.gelu(h)
    o_ref[...] = jnp.dot(h, wv_ref[0], preferred_element_type=jnp.float32)


def _run_ffn(xin, wk, wv, rows_per_block, *, interpret=False):
    N, D = xin.shape
    W, _, H = wk.shape
    nblocks = N // rows_per_block
    per_w = nblocks // W
    out = pl.pallas_call(
        _ffn_body,
        grid=(nblocks,),
        in_specs=[
            pl.BlockSpec((rows_per_block, D), lambda i: (i, 0)),
            pl.BlockSpec((1, D, H), lambda i: (i // per_w, 0, 0)),
            pl.BlockSpec((1, H, D), lambda i: (i // per_w, 0, 0)),
        ],
        out_specs=pl.BlockSpec((rows_per_block, D), lambda i: (i, 0)),
        out_shape=jax.ShapeDtypeStruct((N, D), jnp.float32),
        interpret=interpret,
    )(xin, wk, wv)
    return out


def _combine_sc(yexp, ysh, cg, gates, *, interpret=False):
    N, D = ysh.shape
    per_w = N // _NW
    chunk = 16
    R = 2
    mesh = plsc.VectorSubcoreMesh(
        core_axis_name="c", subcore_axis_name="s",
        num_cores=_NC, num_subcores=_NS)

    ring_scratch = []
    for _ in range(R):
        ring_scratch += [pltpu.VMEM((chunk, D), jnp.float32)] * 3
        ring_scratch += [pltpu.SemaphoreType.DMA] * 2

    @functools.partial(
        pl.kernel, mesh=mesh, interpret=interpret,
        compiler_params=pltpu.CompilerParams(needs_layout_passes=False),
        out_type=jax.ShapeDtypeStruct((N, D), jnp.float32),
        scratch_types=[
            pltpu.VMEM((per_w,), jnp.int32),
            pltpu.VMEM((per_w,), jnp.int32),
            pltpu.VMEM((per_w,), jnp.float32),
            pltpu.VMEM((per_w,), jnp.float32),
        ] + ring_scratch,
    )
    def k(yexp_hbm, ysh_hbm, cg_hbm, gates_hbm, out_hbm,
          i0_v, i1_v, g0_v, g1_v, *ring):
        wid = lax.axis_index("s") * _NC + lax.axis_index("c")
        t0 = wid * per_w
        nch = per_w // chunk
        pltpu.sync_copy(cg_hbm.at[0, pl.ds(t0, per_w)], i0_v)
        pltpu.sync_copy(cg_hbm.at[1, pl.ds(t0, per_w)], i1_v)
        pltpu.sync_copy(gates_hbm.at[0, pl.ds(t0, per_w)], g0_v)
        pltpu.sync_copy(gates_hbm.at[1, pl.ds(t0, per_w)], g1_v)

        sets = tuple(tuple(ring[5 * q:5 * q + 5]) for q in range(R))

        def start_fetch(j, y0, y1, acc, sg):
            b = j * chunk
            pltpu.async_copy(yexp_hbm.at[i0_v.at[pl.ds(b, chunk)]], y0, sg)
            pltpu.async_copy(yexp_hbm.at[i1_v.at[pl.ds(b, chunk)]], y1, sg)
            pltpu.async_copy(ysh_hbm.at[pl.ds(t0 + b, chunk)], acc, sg)

        def drain_fetch(y0, y1, acc, sg):
            pltpu.make_async_copy(ysh_hbm.at[pl.ds(0, chunk)], y0, sg).wait()
            pltpu.make_async_copy(ysh_hbm.at[pl.ds(0, chunk)], y1, sg).wait()
            pltpu.make_async_copy(ysh_hbm.at[pl.ds(0, chunk)], acc, sg).wait()

        def drain_out(acc, so):
            pltpu.make_async_copy(
                acc, out_hbm.at[pl.ds(t0, chunk)], so).wait()

        def do_chunk(j, y0, y1, acc, sg, so):
            drain_fetch(y0, y1, acc, sg)
            b = j * chunk

            def tok(i, _):
                s0 = plsc.load_gather(g0_v, [jnp.full((_L,), b + i, jnp.int32)])
                s1 = plsc.load_gather(g1_v, [jnp.full((_L,), b + i, jnp.int32)])

                def vec(w, _):
                    sl = pl.ds(w * _L, _L)
                    acc[i, sl] = (acc[i, sl] + s0 * y0[i, sl]
                                  + s1 * y1[i, sl])
                    return ()
                lax.fori_loop(0, D // _L, vec, (), unroll=8)
                return ()
            lax.fori_loop(0, chunk, tok, ())
            pltpu.async_copy(acc, out_hbm.at[pl.ds(t0 + b, chunk)], so)

        for q in range(R - 1):
            start_fetch(q, sets[q][0], sets[q][1], sets[q][2], sets[q][3])

        def pipe(jj, _):
            for p, (y0, y1, acc, sg, so) in enumerate(sets):
                j = jj + p
                nxt = sets[(p + R - 1) % R]

                @pl.when(j + R - 1 < nch)
                def _prefetch():
                    @pl.when(j >= 1)
                    def _reclaim():
                        drain_out(nxt[2], nxt[4])
                    start_fetch(j + R - 1, nxt[0], nxt[1], nxt[2], nxt[3])

                do_chunk(j, y0, y1, acc, sg, so)
            return ()
        lax.fori_loop(0, nch // R, lambda q, c: pipe(q * R, c), ())
        for q in range(R):
            drain_out(sets[q][2], sets[q][4])

    return k(yexp, ysh, cg, gates)


def kernel(x, gate_weight, gate_bias, ff_keys, ff_values,
           shared_keys, shared_values):
    B, S_in, D = x.shape
    E = gate_weight.shape[1]
    group_size = min(S_in, 4096)
    G = (B * S_in) // group_size
    S = group_size
    C = _capacity(S, E)
    N = G * S
    NSLOT = G * E * C

    xg = x.reshape(G, S, D)
    idx, gates, cnt = _run_router(xg, gate_weight, gate_bias)

    per_w = N // _NW
    chunk = 32
    nch = per_w // chunk
    ds = jnp.transpose(idx[:, :, 0:2].reshape(_NW, nch, chunk, 2),
                       (0, 3, 1, 2))
    cg = idx[:, :, 2:4].reshape(N, 2).T
    gk = gates[:, :, 0:2].reshape(N, 2).T
    cnt32 = cnt[0]

    x_flat = x.reshape(N, D)
    xin0, xin1 = _scatter_dispatch_sc(x_flat, ds, NSLOT=NSLOT)

    yexp = _run_expert_ffn(cnt32, xin0, xin1, ff_keys, ff_values, G=G, C=C)
    ysh = _run_ffn(x_flat, shared_keys, shared_values, rows_per_block=512)

    out = _combine_sc(yexp, ysh, cg, gk)
    return out.reshape(B, S_in, D)
